# baseline (speedup 1.0000x reference)
import functools

import jax
import jax.numpy as jnp
from jax import lax
from jax.experimental import pallas as pl
from jax.experimental.pallas import tpu as pltpu
from jax.experimental.pallas import tpu_sc as plsc

N = 100000
D = 128
NUM_CORES = 2
NUM_SUBCORES = 16
NUM_WORKERS = NUM_CORES * NUM_SUBCORES
PER_W = 3136
LAST_BASE = N - PER_W
CHUNK = 224
NCHUNK = PER_W // CHUNK
NBUF = 3

_mesh = plsc.VectorSubcoreMesh(core_axis_name="c", subcore_axis_name="s")


@functools.partial(
    pl.kernel,
    mesh=_mesh,
    out_type=jax.ShapeDtypeStruct((N, D), jnp.float32),
    scratch_types=[
        pltpu.VMEM((PER_W,), jnp.int32),
        pltpu.VMEM((NBUF, CHUNK, D), jnp.float32),
        pltpu.VMEM_SHARED((94, D), jnp.float32),
        pltpu.SemaphoreType.DMA,
        pltpu.SemaphoreType.DMA,
        pltpu.SemaphoreType.DMA,
        pltpu.SemaphoreType.DMA,
        pltpu.SemaphoreType.DMA,
        pltpu.SemaphoreType.DMA,
        pltpu.SemaphoreType.DMA,
    ],
)
def _emb_lookup(idx_hbm, table_hbm, out_hbm, idx_v, rows_v, table_sh, isem,
                gsem0, gsem1, gsem2, ssem0, ssem1, ssem2):
    gsems = (gsem0, gsem1, gsem2)
    ssems = (ssem0, ssem1, ssem2)
    sid = lax.axis_index("s")
    wid = sid * NUM_CORES + lax.axis_index("c")
    base = jnp.minimum(wid * PER_W, LAST_BASE)
    @pl.when(sid == 0)
    def _():
        pltpu.sync_copy(table_hbm, table_sh)

    pltpu.async_copy(idx_hbm.at[pl.ds(base, PER_W)], idx_v, isem).wait()
    plsc.subcore_barrier()

    def gather(j):
        src = table_sh if j % 2 == 0 else table_hbm
        return pltpu.async_copy(
            src.at[idx_v.at[pl.ds(j * CHUNK, CHUNK)]],
            rows_v.at[j % NBUF],
            gsems[j % NBUF],
        )

    gathers = {j: gather(j) for j in range(NBUF - 1)}
    stores = {}
    for j in range(NCHUNK):
        jn = j + NBUF - 1
        if jn < NCHUNK:
            if jn >= NBUF:
                stores[jn - NBUF].wait()
            gathers[jn] = gather(jn)
        gathers[j].wait()
        stores[j] = pltpu.async_copy(
            rows_v.at[j % NBUF],
            out_hbm.at[pl.ds(base + j * CHUNK, CHUNK)],
            ssems[j % NBUF],
        )
    for j in range(max(0, NCHUNK - NBUF), NCHUNK):
        stores[j].wait()


def kernel(atomic_numbers, embedding_weight):
    return _emb_lookup(atomic_numbers.astype(jnp.int32), embedding_weight)


# speedup vs baseline: 2.6315x; 2.6315x over previous
import functools

import jax
import jax.numpy as jnp
from jax import lax
from jax.experimental import pallas as pl
from jax.experimental.pallas import tpu as pltpu
from jax.experimental.pallas import tpu_sc as plsc

N = 100000
D = 128
NUM_CORES = 2
NUM_SUBCORES = 16
NUM_WORKERS = NUM_CORES * NUM_SUBCORES
PER_W = 3136
LAST_BASE = N - PER_W
CHUNK = 112
NCHUNK = PER_W // CHUNK
NBUF = 6

_mesh = plsc.VectorSubcoreMesh(core_axis_name="c", subcore_axis_name="s")


@functools.partial(
    pl.kernel,
    mesh=_mesh,
    out_type=jax.ShapeDtypeStruct((N, D), jnp.float32),
    scratch_types=[
        pltpu.VMEM((PER_W,), jnp.int32),
        pltpu.VMEM((NBUF, CHUNK, D), jnp.float32),
        pltpu.VMEM_SHARED((94, D), jnp.float32),
        pltpu.SemaphoreType.DMA,
        pltpu.SemaphoreType.DMA,
        pltpu.SemaphoreType.DMA,
        pltpu.SemaphoreType.DMA,
        pltpu.SemaphoreType.DMA,
        pltpu.SemaphoreType.DMA,
        pltpu.SemaphoreType.DMA,
        pltpu.SemaphoreType.DMA,
        pltpu.SemaphoreType.DMA,
        pltpu.SemaphoreType.DMA,
        pltpu.SemaphoreType.DMA,
        pltpu.SemaphoreType.DMA,
        pltpu.SemaphoreType.DMA,
    ],
)
def _emb_lookup(idx_hbm, table_hbm, out_hbm, idx_v, rows_v, table_sh, isem,
                gsem0, gsem1, gsem2, gsem3, gsem4, gsem5,
                ssem0, ssem1, ssem2, ssem3, ssem4, ssem5):
    gsems = (gsem0, gsem1, gsem2, gsem3, gsem4, gsem5)
    ssems = (ssem0, ssem1, ssem2, ssem3, ssem4, ssem5)
    sid = lax.axis_index("s")
    wid = sid * NUM_CORES + lax.axis_index("c")
    base = jnp.minimum(wid * PER_W, LAST_BASE)
    @pl.when(sid == 0)
    def _():
        pltpu.sync_copy(table_hbm, table_sh)

    pltpu.async_copy(idx_hbm.at[pl.ds(base, PER_W)], idx_v, isem).wait()
    plsc.subcore_barrier()

    def gather(j):
        return pltpu.async_copy(
            table_sh.at[idx_v.at[pl.ds(j * CHUNK, CHUNK)]],
            rows_v.at[j % NBUF],
            gsems[j % NBUF],
        )

    gathers = {j: gather(j) for j in range(NBUF - 1)}
    stores = {}
    for j in range(NCHUNK):
        jn = j + NBUF - 1
        if jn < NCHUNK:
            if jn >= NBUF:
                stores[jn - NBUF].wait()
            gathers[jn] = gather(jn)
        gathers[j].wait()
        stores[j] = pltpu.async_copy(
            rows_v.at[j % NBUF],
            out_hbm.at[pl.ds(base + j * CHUNK, CHUNK)],
            ssems[j % NBUF],
        )
    for j in range(max(0, NCHUNK - NBUF), NCHUNK):
        stores[j].wait()


def kernel(atomic_numbers, embedding_weight):
    return _emb_lookup(atomic_numbers.astype(jnp.int32), embedding_weight)


# speedup vs baseline: 2.6880x; 1.0215x over previous
import functools

import jax
import jax.numpy as jnp
from jax import lax
from jax.experimental import pallas as pl
from jax.experimental.pallas import tpu as pltpu
from jax.experimental.pallas import tpu_sc as plsc

N = 100000
D = 128
NUM_CORES = 2
NUM_SUBCORES = 16
NUM_WORKERS = NUM_CORES * NUM_SUBCORES
PER_W = 3136
LAST_BASE = N - PER_W
CHUNK = 224
NCHUNK = PER_W // CHUNK
NBUF = 3

_mesh = plsc.VectorSubcoreMesh(core_axis_name="c", subcore_axis_name="s")


@functools.partial(
    pl.kernel,
    mesh=_mesh,
    out_type=jax.ShapeDtypeStruct((N, D), jnp.float32),
    scratch_types=[
        pltpu.VMEM((PER_W,), jnp.int32),
        pltpu.VMEM((NBUF, CHUNK, D), jnp.float32),
        pltpu.VMEM_SHARED((94, D), jnp.float32),
        pltpu.SemaphoreType.DMA,
        pltpu.SemaphoreType.DMA,
        pltpu.SemaphoreType.DMA,
        pltpu.SemaphoreType.DMA,
        pltpu.SemaphoreType.DMA,
        pltpu.SemaphoreType.DMA,
        pltpu.SemaphoreType.DMA,
    ],
)
def _emb_lookup(idx_hbm, table_hbm, out_hbm, idx_v, rows_v, table_sh, isem,
                gsem0, gsem1, gsem2, ssem0, ssem1, ssem2):
    gsems = (gsem0, gsem1, gsem2)
    ssems = (ssem0, ssem1, ssem2)
    sid = lax.axis_index("s")
    wid = sid * NUM_CORES + lax.axis_index("c")
    base = jnp.minimum(wid * PER_W, LAST_BASE)
    @pl.when(sid == 0)
    def _():
        pltpu.sync_copy(table_hbm, table_sh)

    pltpu.async_copy(idx_hbm.at[pl.ds(base, PER_W)], idx_v, isem).wait()
    plsc.subcore_barrier()

    def gather(j):
        return pltpu.async_copy(
            table_sh.at[idx_v.at[pl.ds(j * CHUNK, CHUNK)]],
            rows_v.at[j % NBUF],
            gsems[j % NBUF],
        )

    gathers = {j: gather(j) for j in range(NBUF - 1)}
    stores = {}
    for j in range(NCHUNK):
        jn = j + NBUF - 1
        if jn < NCHUNK:
            if jn >= NBUF:
                stores[jn - NBUF].wait()
            gathers[jn] = gather(jn)
        gathers[j].wait()
        stores[j] = pltpu.async_copy(
            rows_v.at[j % NBUF],
            out_hbm.at[pl.ds(base + j * CHUNK, CHUNK)],
            ssems[j % NBUF],
        )
    for j in range(max(0, NCHUNK - NBUF), NCHUNK):
        stores[j].wait()


def kernel(atomic_numbers, embedding_weight):
    return _emb_lookup(atomic_numbers.astype(jnp.int32), embedding_weight)
